# baseline (speedup 1.0000x reference)
import functools

import numpy as np

import jax
import jax.numpy as jnp
from jax import lax
from jax.experimental import pallas as pl
from jax.experimental.pallas import tpu as pltpu
from jax.experimental.pallas import tpu_sc as plsc

_N = 10000
_M = 128
_MH = 64
_MY = 10
_GAMMA = 0.8
_ITERS = 30
_EPS_F = 1e-6

_NC, _NS = 2, 16
_CHUNK = 125
_NP = 10240
_ROWS_PER_TILE = _NP // _NS

_mesh = plsc.VectorSubcoreMesh(core_axis_name="c", subcore_axis_name="s")


def _sc_segment_sum(zs2, zeros, rx3, c3):
    n_chunks = c3.shape[1]
    nq = n_chunks // 4
    assert n_chunks % 4 == 0 and rx3.shape == (_NC, _NS, n_chunks, _CHUNK)

    @functools.partial(
        pl.kernel,
        out_type=jax.ShapeDtypeStruct((_NC, _NP, _MH), jnp.float32),
        mesh=_mesh,
        scratch_types=[
            pltpu.VMEM((n_chunks, _CHUNK), jnp.int32),
            pltpu.VMEM((n_chunks, _CHUNK), jnp.int32),
            [pltpu.VMEM((_CHUNK, _MH), jnp.bfloat16)] * 4,
            [pltpu.VMEM((_CHUNK, _MH), jnp.float32)] * 2,
            pltpu.VMEM_SHARED((_NP, _MH), jnp.float32),
            [pltpu.SemaphoreType.DMA] * 4,
            [pltpu.SemaphoreType.DMA] * 2,
        ],
        compiler_params=pltpu.CompilerParams(use_tc_tiling_on_sc=False,
                                             needs_layout_passes=False),
    )
    def run(zs_hbm, zeros_hbm, rx3_hbm, c3_hbm, out_hbm,
            ribuf, cibuf, braw, frows, acc_sh, gsem, ssem):
        cid = lax.axis_index("c")
        sid = lax.axis_index("s")

        my_rows = pl.ds(sid * _ROWS_PER_TILE, _ROWS_PER_TILE)
        pltpu.sync_copy(rx3_hbm.at[cid, sid], ribuf)
        pltpu.sync_copy(c3_hbm.at[sid], cibuf)
        pltpu.sync_copy(zeros_hbm, acc_sh.at[my_rows])
        plsc.subcore_barrier()

        def fire(g, b):
            pltpu.async_copy(zs_hbm.at[ribuf.at[g]], braw[b], gsem[b])

        def gwait(b):
            pltpu.make_async_copy(zs_hbm.at[ribuf.at[0]], braw[b],
                                  gsem[b]).wait()

        def scat(g, f):
            pltpu.async_copy(frows[f], acc_sh.at[cibuf.at[g]], ssem[f],
                             add=True)

        def scwait(f):
            pltpu.make_async_copy(zeros_hbm.at[pl.ds(0, _CHUNK)], frows[f],
                                  ssem[f]).wait()

        def conv(b, f):
            def crow(k, carry):
                for h in range(2):
                    v = braw[b][k, pl.ds(32 * h, 32)]
                    lo, hi = plsc.unpack(
                        v, format=plsc.PackFormat.INTERLEAVED)
                    frows[f][k, pl.ds(32 * h, 16)] = lo
                    frows[f][k, pl.ds(32 * h + 16, 16)] = hi
                return carry
            lax.fori_loop(0, _CHUNK, crow, 0)

        for b in range(4):
            fire(b, b)
        for b in range(4):
            gwait(b)
            if b >= 2:
                scwait(b % 2)
            conv(b, b % 2)
            fire(b + 4, b)
            scat(b, b % 2)

        def body(p, carry):
            q = 4 * p
            for b in range(4):
                gwait(b)
                scwait(b % 2)
                conv(b, b % 2)
                fire(q + b + 4, b)
                scat(q + b, b % 2)
            return carry

        lax.fori_loop(1, nq - 1, body, 0)

        q = n_chunks - 4
        for b in range(4):
            gwait(b)
            scwait(b % 2)
            conv(b, b % 2)
            scat(q + b, b % 2)
        for f in range(2):
            scwait(f)

        plsc.subcore_barrier()
        pltpu.sync_copy(acc_sh.at[my_rows], out_hbm.at[cid, my_rows])

    return run(zs2, zeros, rx3, c3)


def _sc_degree(ones, zeros, r3, c3):
    n_chunks = r3.shape[1]

    @functools.partial(
        pl.kernel,
        out_type=jax.ShapeDtypeStruct((_NC, _NP, _MH), jnp.float32),
        mesh=_mesh,
        scratch_types=[
            pltpu.VMEM((n_chunks, _CHUNK), jnp.int32),
            pltpu.VMEM((n_chunks, _CHUNK), jnp.int32),
            pltpu.VMEM((_CHUNK, _MH), jnp.float32),
            pltpu.VMEM_SHARED((_NP, _MH), jnp.float32),
        ],
        compiler_params=pltpu.CompilerParams(use_tc_tiling_on_sc=False),
    )
    def run(ones_hbm, zeros_hbm, r3_hbm, c3_hbm, out_hbm,
            ribuf, cibuf, ones_v, acc_sh):
        cid = lax.axis_index("c")
        sid = lax.axis_index("s")

        my_rows = pl.ds(sid * _ROWS_PER_TILE, _ROWS_PER_TILE)
        pltpu.sync_copy(r3_hbm.at[sid], ribuf)
        pltpu.sync_copy(c3_hbm.at[sid], cibuf)
        pltpu.sync_copy(zeros_hbm, acc_sh.at[my_rows])
        pltpu.sync_copy(ones_hbm, ones_v)
        plsc.subcore_barrier()

        def step(g, carry):
            pltpu.sync_copy(ones_v, acc_sh.at[ribuf.at[g]], add=True)
            pltpu.sync_copy(ones_v, acc_sh.at[cibuf.at[g]], add=True)
            return carry

        lax.fori_loop(0, n_chunks, step, 0)
        plsc.subcore_barrier()
        pltpu.sync_copy(acc_sh.at[my_rows], out_hbm.at[cid, my_rows])

    return run(ones, zeros, r3, c3)


_BR = 1024


def _tc_prep(pdeg, xt):
    def body(p_ref, xt_ref, d2_ref, xd_ref):
        deg = p_ref[0][:, 0:1]
        d = lax.rsqrt(jnp.maximum(deg, 1.0))
        d2_ref[...] = d * d
        xd_ref[...] = xt_ref[...] * d

    return pl.pallas_call(
        body,
        grid=(_NP // _BR,),
        in_specs=[
            pl.BlockSpec((_NC, _BR, _MH), lambda i: (0, i, 0)),
            pl.BlockSpec((_BR, _M), lambda i: (i, 0)),
        ],
        out_specs=[
            pl.BlockSpec((_BR, 1), lambda i: (i, 0)),
            pl.BlockSpec((_BR, _M), lambda i: (i, 0)),
        ],
        out_shape=[
            jax.ShapeDtypeStruct((_NP, 1), jnp.float32),
            jax.ShapeDtypeStruct((_NP, _M), jnp.float32),
        ],
    )(pdeg, xt)


def _tc_gmat(f):
    def body(f_ref, g_ref):
        fm = f_ref[...]
        ff = lax.dot_general(fm, fm, (((0,), (0,)), ((), ())),
                             preferred_element_type=jnp.float32)
        nrm = jnp.sqrt(jnp.sum(ff * ff))
        g_ref[...] = (_GAMMA / (nrm + _EPS_F)) * ff

    return pl.pallas_call(
        body,
        out_shape=jax.ShapeDtypeStruct((_M, _M), jnp.float32),
    )(f)


def _tc_update(halves, d2, xdp, ggp, out_dtype=jnp.bfloat16):
    def body(p_ref, d2_ref, xd_ref, gg_ref, o_ref):
        acc = jnp.concatenate([p_ref[0], p_ref[1]], axis=1)
        y = acc * d2_ref[...]
        zn = (
            jnp.dot(y, gg_ref[...], preferred_element_type=jnp.float32)
            + xd_ref[...]
        )
        o_ref[...] = zn.astype(o_ref.dtype)

    return pl.pallas_call(
        body,
        grid=(_NP // _BR,),
        in_specs=[
            pl.BlockSpec((_NC, _BR, _MH), lambda i: (0, i, 0)),
            pl.BlockSpec((_BR, 1), lambda i: (i, 0)),
            pl.BlockSpec((_BR, _M), lambda i: (i, 0)),
            pl.BlockSpec((_M, _M), lambda i: (0, 0)),
        ],
        out_specs=pl.BlockSpec((_BR, _M), lambda i: (i, 0)),
        out_shape=jax.ShapeDtypeStruct((_NP, _M), out_dtype),
    )(halves, d2, xdp, ggp)


def _tc_head(zs, bp):
    def body(z_ref, b_ref, o_ref):
        z = z_ref[...].astype(jnp.float32)
        nrm = jnp.maximum(jnp.sqrt(jnp.sum(z * z, axis=1, keepdims=True)), 1e-12)
        o_ref[...] = jnp.dot(z / nrm, b_ref[...],
                             preferred_element_type=jnp.float32)

    return pl.pallas_call(
        body,
        grid=(_NP // _BR,),
        in_specs=[
            pl.BlockSpec((_BR, _M), lambda i: (i, 0)),
            pl.BlockSpec((_M, _M), lambda i: (0, 0)),
        ],
        out_specs=pl.BlockSpec((_BR, _M), lambda i: (i, 0)),
        out_shape=jax.ShapeDtypeStruct((_NP, _M), jnp.float32),
    )(zs, bp)


@jax.jit
def _run(X, edge_index, F_param, B_W):
    e = edge_index.shape[1]
    n_chunks = e // (_NS * _CHUNK)
    r3 = edge_index[0].astype(jnp.int32).reshape(_NS, n_chunks, _CHUNK)
    c3 = edge_index[1].astype(jnp.int32).reshape(_NS, n_chunks, _CHUNK)
    rx3 = jnp.stack([2 * r3, 2 * r3 + 1])
    xt = jnp.pad(X.T.astype(jnp.float32), ((0, _NP - _N), (0, 0)))
    zeros = jnp.zeros((_ROWS_PER_TILE, _MH), jnp.float32)
    ones = jnp.ones((_CHUNK, _MH), jnp.float32)
    bp = jnp.pad(B_W.T.astype(jnp.float32), ((0, 0), (0, _M - _MY)))

    pdeg = _sc_degree(ones, zeros, r3, c3)
    d2, xd = _tc_prep(pdeg, xt)
    gg = _tc_gmat(F_param.astype(jnp.float32))

    perm = np.arange(_M).reshape(-1, 2, 16).transpose(0, 2, 1).reshape(_M)
    ggp = gg[:, perm]
    xdp = xd[:, perm]
    bpp = bp[perm, :]

    def it(zs, _):
        zs2 = zs.reshape(2 * _NP, _MH)
        h = _sc_segment_sum(zs2, zeros, rx3, c3)
        return _tc_update(h, d2, xdp, ggp), None

    zs, _ = lax.scan(it, xdp.astype(jnp.bfloat16), None, length=_ITERS)
    out = _tc_head(zs, bpp)
    return out[:_N, :_MY]


def kernel(X, edge_index, F_param, B_W):
    return _run(X, edge_index, F_param, B_W)


# speedup vs baseline: 1.7216x; 1.7216x over previous
import functools

import jax
import jax.numpy as jnp
from jax import lax
from jax.experimental import pallas as pl
from jax.experimental.pallas import tpu as pltpu
from jax.experimental.pallas import tpu_sc as plsc

_N = 10000
_M = 128
_MH = 64
_MY = 10
_GAMMA = 0.8
_ITERS = 30
_EPS_F = 1e-6

_NC, _NS = 2, 16
_CHUNK = 125
_NP = 10240
_ROWS_PER_TILE = _NP // _NS

_mesh = plsc.VectorSubcoreMesh(core_axis_name="c", subcore_axis_name="s")


def _sc_segment_sum(zs2, zeros, rx3, c3):
    n_chunks = c3.shape[1]
    nq = n_chunks // 4
    assert n_chunks % 4 == 0 and rx3.shape == (_NC, _NS, n_chunks, _CHUNK)

    @functools.partial(
        pl.kernel,
        out_type=jax.ShapeDtypeStruct((_NC, _NP, _MH), jnp.float32),
        mesh=_mesh,
        scratch_types=[
            pltpu.VMEM((n_chunks, _CHUNK), jnp.int32),
            pltpu.VMEM((n_chunks, _CHUNK), jnp.int32),
            [pltpu.VMEM((_CHUNK, _MH), jnp.float32)] * 4,
            pltpu.VMEM_SHARED((_NP, _MH), jnp.float32),
            [pltpu.SemaphoreType.DMA] * 4,
            [pltpu.SemaphoreType.DMA] * 4,
        ],
        compiler_params=pltpu.CompilerParams(use_tc_tiling_on_sc=False),
    )
    def run(zs_hbm, zeros_hbm, rx3_hbm, c3_hbm, out_hbm,
            ribuf, cibuf, rows, acc_sh, gsem, ssem):
        cid = lax.axis_index("c")
        sid = lax.axis_index("s")

        my_rows = pl.ds(sid * _ROWS_PER_TILE, _ROWS_PER_TILE)
        pltpu.sync_copy(rx3_hbm.at[cid, sid], ribuf)
        pltpu.sync_copy(c3_hbm.at[sid], cibuf)
        pltpu.sync_copy(zeros_hbm, acc_sh.at[my_rows])
        plsc.subcore_barrier()

        def fire(g, b):
            pltpu.async_copy(zs_hbm.at[ribuf.at[g]], rows[b], gsem[b])

        def gwait(b):
            pltpu.make_async_copy(zs_hbm.at[ribuf.at[0]], rows[b],
                                  gsem[b]).wait()

        def scat(g, b):
            pltpu.async_copy(rows[b], acc_sh.at[cibuf.at[g]], ssem[b],
                             add=True)

        def scwait(b):
            pltpu.make_async_copy(zs_hbm.at[ribuf.at[0]], rows[b],
                                  ssem[b]).wait()

        for b in range(4):
            fire(b, b)
        for b in range(4):
            gwait(b)
            scat(b, b)

        def body(p, carry):
            q = 4 * p
            for b in range(4):
                scwait(b)
                fire(q + b, b)
            for b in range(4):
                gwait(b)
                scat(q + b, b)
            return carry

        lax.fori_loop(1, nq, body, 0)
        for b in range(4):
            scwait(b)

        plsc.subcore_barrier()
        pltpu.sync_copy(acc_sh.at[my_rows], out_hbm.at[cid, my_rows])

    return run(zs2, zeros, rx3, c3)


def _sc_degree(ones, zeros, r3, c3):
    n_chunks = r3.shape[1]

    @functools.partial(
        pl.kernel,
        out_type=jax.ShapeDtypeStruct((_NC, _NP, _MH), jnp.float32),
        mesh=_mesh,
        scratch_types=[
            pltpu.VMEM((n_chunks, _CHUNK), jnp.int32),
            pltpu.VMEM((n_chunks, _CHUNK), jnp.int32),
            pltpu.VMEM((_CHUNK, _MH), jnp.float32),
            pltpu.VMEM_SHARED((_NP, _MH), jnp.float32),
        ],
        compiler_params=pltpu.CompilerParams(use_tc_tiling_on_sc=False),
    )
    def run(ones_hbm, zeros_hbm, r3_hbm, c3_hbm, out_hbm,
            ribuf, cibuf, ones_v, acc_sh):
        cid = lax.axis_index("c")
        sid = lax.axis_index("s")

        my_rows = pl.ds(sid * _ROWS_PER_TILE, _ROWS_PER_TILE)
        pltpu.sync_copy(r3_hbm.at[sid], ribuf)
        pltpu.sync_copy(c3_hbm.at[sid], cibuf)
        pltpu.sync_copy(zeros_hbm, acc_sh.at[my_rows])
        pltpu.sync_copy(ones_hbm, ones_v)
        plsc.subcore_barrier()

        def step(g, carry):
            pltpu.sync_copy(ones_v, acc_sh.at[ribuf.at[g]], add=True)
            pltpu.sync_copy(ones_v, acc_sh.at[cibuf.at[g]], add=True)
            return carry

        lax.fori_loop(0, n_chunks, step, 0)
        plsc.subcore_barrier()
        pltpu.sync_copy(acc_sh.at[my_rows], out_hbm.at[cid, my_rows])

    return run(ones, zeros, r3, c3)


_BR = 1024


def _tc_prep(pdeg, xt):
    def body(p_ref, xt_ref, d2_ref, xd_ref):
        deg = p_ref[0][:, 0:1]
        d = lax.rsqrt(jnp.maximum(deg, 1.0))
        d2_ref[...] = d * d
        xd_ref[...] = xt_ref[...] * d

    return pl.pallas_call(
        body,
        grid=(_NP // _BR,),
        in_specs=[
            pl.BlockSpec((_NC, _BR, _MH), lambda i: (0, i, 0)),
            pl.BlockSpec((_BR, _M), lambda i: (i, 0)),
        ],
        out_specs=[
            pl.BlockSpec((_BR, 1), lambda i: (i, 0)),
            pl.BlockSpec((_BR, _M), lambda i: (i, 0)),
        ],
        out_shape=[
            jax.ShapeDtypeStruct((_NP, 1), jnp.float32),
            jax.ShapeDtypeStruct((_NP, _M), jnp.float32),
        ],
    )(pdeg, xt)


def _tc_gmat(f):
    def body(f_ref, g_ref):
        fm = f_ref[...]
        ff = lax.dot_general(fm, fm, (((0,), (0,)), ((), ())),
                             preferred_element_type=jnp.float32)
        nrm = jnp.sqrt(jnp.sum(ff * ff))
        g_ref[...] = (_GAMMA / (nrm + _EPS_F)) * ff

    return pl.pallas_call(
        body,
        out_shape=jax.ShapeDtypeStruct((_M, _M), jnp.float32),
    )(f)


def _tc_update(halves, d2, xd, gg):
    def body(p_ref, d2_ref, xd_ref, gg_ref, o_ref):
        acc = jnp.concatenate([p_ref[0], p_ref[1]], axis=1)
        y = acc * d2_ref[...]
        o_ref[...] = (
            jnp.dot(y, gg_ref[...], preferred_element_type=jnp.float32)
            + xd_ref[...]
        )

    return pl.pallas_call(
        body,
        grid=(_NP // _BR,),
        in_specs=[
            pl.BlockSpec((_NC, _BR, _MH), lambda i: (0, i, 0)),
            pl.BlockSpec((_BR, 1), lambda i: (i, 0)),
            pl.BlockSpec((_BR, _M), lambda i: (i, 0)),
            pl.BlockSpec((_M, _M), lambda i: (0, 0)),
        ],
        out_specs=pl.BlockSpec((_BR, _M), lambda i: (i, 0)),
        out_shape=jax.ShapeDtypeStruct((_NP, _M), jnp.float32),
    )(halves, d2, xd, gg)


def _tc_head(zs, bp):
    def body(z_ref, b_ref, o_ref):
        z = z_ref[...]
        nrm = jnp.maximum(jnp.sqrt(jnp.sum(z * z, axis=1, keepdims=True)), 1e-12)
        o_ref[...] = jnp.dot(z / nrm, b_ref[...],
                             preferred_element_type=jnp.float32)

    return pl.pallas_call(
        body,
        grid=(_NP // _BR,),
        in_specs=[
            pl.BlockSpec((_BR, _M), lambda i: (i, 0)),
            pl.BlockSpec((_M, _M), lambda i: (0, 0)),
        ],
        out_specs=pl.BlockSpec((_BR, _M), lambda i: (i, 0)),
        out_shape=jax.ShapeDtypeStruct((_NP, _M), jnp.float32),
    )(zs, bp)


@jax.jit
def _run(X, edge_index, F_param, B_W):
    e = edge_index.shape[1]
    n_chunks = e // (_NS * _CHUNK)
    r3 = edge_index[0].astype(jnp.int32).reshape(_NS, n_chunks, _CHUNK)
    c3 = edge_index[1].astype(jnp.int32).reshape(_NS, n_chunks, _CHUNK)
    rx3 = jnp.stack([2 * r3, 2 * r3 + 1])
    xt = jnp.pad(X.T.astype(jnp.float32), ((0, _NP - _N), (0, 0)))
    zeros = jnp.zeros((_ROWS_PER_TILE, _MH), jnp.float32)
    ones = jnp.ones((_CHUNK, _MH), jnp.float32)
    bp = jnp.pad(B_W.T.astype(jnp.float32), ((0, 0), (0, _M - _MY)))

    pdeg = _sc_degree(ones, zeros, r3, c3)
    d2, xd = _tc_prep(pdeg, xt)
    gg = _tc_gmat(F_param.astype(jnp.float32))

    def it(zs, _):
        zs2 = zs.reshape(2 * _NP, _MH)
        h = _sc_segment_sum(zs2, zeros, rx3, c3)
        return _tc_update(h, d2, xd, gg), None

    zs, _ = lax.scan(it, xd, None, length=_ITERS)
    out = _tc_head(zs, bp)
    return out[:_N, :_MY]


def kernel(X, edge_index, F_param, B_W):
    return _run(X, edge_index, F_param, B_W)
